# Initial kernel scaffold; baseline (speedup 1.0000x reference)
import jax
import jax.numpy as jnp
from jax.experimental import pallas as pl


def kernel(msa, deletion_matrix, aatype):
    raise NotImplementedError("write your pallas kernel here")


# speedup vs baseline: 2.3418x; 2.3418x over previous
import functools
import math

import jax
import jax.numpy as jnp
from jax.experimental import pallas as pl
from jax.experimental.pallas import tpu as pltpu

NUM_MSA_C = 4096
NUM_RES_C = 256
NUM_CLUST_C = 512
NUM_EXTRA_C = 1024
NUM_RECYCLE_C = 1
MASK_TOKEN_C = 22

_F32 = jnp.float32
_I32 = jnp.int32
_HI = jax.lax.Precision.HIGHEST


def _atan_pos(y):
    inv = y > 1.0
    t = jnp.where(inv, 1.0 / jnp.maximum(y, 1e-30), y)
    t = t / (1.0 + jnp.sqrt(1.0 + t * t))
    t = t / (1.0 + jnp.sqrt(1.0 + t * t))
    t2 = t * t
    p = t * (1.0 + t2 * (-1.0 / 3.0 + t2 * (0.2 + t2 * (-1.0 / 7.0 + t2 / 9.0))))
    p = 4.0 * p
    return jnp.where(inv, (math.pi / 2.0) - p, p)


def _profile_kernel(msa_ref, aat_ref, cnt_ref, tf_ref):
    m = msa_ref[...]
    cols = [jnp.sum((m == c).astype(_F32), axis=0, keepdims=True) for c in range(22)]
    cnt_ref[...] = jnp.concatenate(cols, axis=1)
    aat = aat_ref[...]
    cls = jax.lax.broadcasted_iota(_I32, (NUM_RES_C, 22), 1) - 1
    tf_ref[...] = (aat == cls).astype(_F32)


def _gather_kernel(gidx_ref, msa_ref, del_ref, msa_out_ref, del_out_ref):
    idx = gidx_ref[...]
    cols = jax.lax.broadcasted_iota(_I32, (idx.shape[0], NUM_MSA_C), 1)
    p = (idx == cols)
    msa_out_ref[...] = jnp.dot(p.astype(jnp.bfloat16), msa_ref[...].astype(jnp.bfloat16),
                               preferred_element_type=_F32).astype(_I32)
    del_out_ref[...] = jnp.dot(p.astype(_F32), del_ref[...],
                               preferred_element_type=_F32, precision=_HI)


def _bert_kernel(mc_ref, dc_ref, logits_ref, gum_ref, mask_ref, rc_ref, ur_ref,
                 bert_ref, bmask_ref, hasdel_ref, delval_ref):
    x = gum_ref[...] + logits_ref[...][None]
    maxv = jnp.max(x, axis=1, keepdims=True)
    citer = jax.lax.broadcasted_iota(_I32, x.shape, 1).astype(_F32)
    ps = jnp.min(jnp.where(x == maxv, citer, 22.0), axis=1).astype(_I32)
    mc = mc_ref[...]
    rc = rc_ref[...]
    mv = jnp.where(rc < 0.1, ur_ref[...],
         jnp.where(rc < 0.2, ps,
         jnp.where(rc < 0.3, mc, MASK_TOKEN_C)))
    mask = mask_ref[...]
    bert_ref[...] = jnp.where(mask != 0, mv, mc)
    bmask_ref[...] = mask.astype(_F32)
    dc = dc_ref[...]
    hasdel_ref[...] = (dc > 0.0).astype(_F32)
    delval_ref[...] = _atan_pos(dc / 3.0) * (2.0 / math.pi)


def _main_kernel(msa_ref, del_ref, bert_ref, isx_ref, dc_ref,
                 s2d_ref, cnts_ref, dmv_ref,
                 a_scr, s_scr, dsum_scr, cnt_scr):
    j = pl.program_id(0)
    nj = pl.num_programs(0)

    @pl.when(j == 0)
    def _init():
        b = bert_ref[...]
        a_scr[...] = jnp.concatenate(
            [(b == c) for c in range(21)], axis=1).astype(jnp.bfloat16)
        s_scr[...] = jnp.zeros_like(s_scr)
        dsum_scr[...] = jnp.zeros_like(dsum_scr)
        cnt_scr[...] = jnp.zeros_like(cnt_scr)

    m = msa_ref[...]
    b23 = jnp.concatenate([(m == c) for c in range(23)], axis=1).astype(jnp.bfloat16)
    scores_t = jax.lax.dot_general(
        a_scr[...], b23[:, :21 * NUM_RES_C], (((1,), (1,)), ((), ())),
        preferred_element_type=_F32)
    maxv = jnp.max(scores_t, axis=0, keepdims=True)
    miota = jax.lax.broadcasted_iota(_I32, scores_t.shape, 0).astype(_F32)
    am_t = jnp.min(jnp.where(scores_t == maxv, miota, float(NUM_CLUST_C)),
                   axis=0, keepdims=True)
    cmat = (am_t == miota).astype(_F32) * isx_ref[0]
    cnt_scr[...] += jnp.sum(cmat, axis=1, keepdims=True)
    s_scr[...] += jax.lax.dot_general(
        cmat.astype(jnp.bfloat16), b23, (((1,), (0,)), ((), ())),
        preferred_element_type=_F32)
    dsum_scr[...] += jax.lax.dot_general(
        cmat, del_ref[...], (((1,), (0,)), ((), ())),
        preferred_element_type=_F32, precision=_HI)

    @pl.when(j == nj - 1)
    def _fini():
        cnts = cnt_scr[...] + 1.0
        s2d_ref[...] = s_scr[...]
        cnts_ref[...] = cnts
        dmean = (dsum_scr[...] + dc_ref[...]) / cnts
        dmv_ref[...] = _atan_pos(dmean / 3.0) * (2.0 / math.pi)


def _feat_kernel(s2d_ref, bert_ref, cnts_ref, prof_ref, samp_ref):
    c = pl.program_id(0)
    samp = (bert_ref[...] == c).astype(_F32)
    samp_ref[...] = samp
    prof_ref[...] = (s2d_ref[...] + samp) / cnts_ref[...]


def kernel(msa, deletion_matrix, aatype):
    num_msa, num_res = msa.shape
    nc, nx = NUM_CLUST_C, NUM_EXTRA_C
    key = jax.random.key(7)

    cnt2d, target_feat_oh = pl.pallas_call(
        _profile_kernel,
        out_shape=(jax.ShapeDtypeStruct((1, 22 * num_res), _F32),
                   jax.ShapeDtypeStruct((num_res, 22), _F32)),
    )(msa, aatype.reshape(num_res, 1))
    profile_logits_t = jnp.log(cnt2d / num_msa + 1e-6).reshape(22, num_res)

    msa_feats, bert_masks, true_msas, extra_msas, extra_dels = [], [], [], [], []
    for it in range(NUM_RECYCLE_C + 1):
        ki = jax.random.fold_in(key, it)
        perm_rest = 1 + jax.random.permutation(jax.random.fold_in(ki, 0), num_msa - 1)
        order = jnp.concatenate([jnp.zeros((1,), perm_rest.dtype), perm_rest])
        sel = order[:nc]
        unsel = order[nc:]
        mask_pos = (jax.random.uniform(jax.random.fold_in(ki, 1), (nc, num_res)) < 0.15)
        rand_cat = jax.random.uniform(jax.random.fold_in(ki, 2), (nc, num_res))
        uniform_repl = jax.random.randint(jax.random.fold_in(ki, 3), (nc, num_res), 0, 20)
        gumbel_t = jnp.transpose(
            jax.random.gumbel(jax.random.fold_in(ki, 4), (nc, num_res, 22), _F32),
            (0, 2, 1))
        crop_idx = jax.random.permutation(jax.random.fold_in(ki, 5), num_msa - nc)[:nx]
        extra_sel = unsel[crop_idx]
        is_extra = jnp.ones((num_msa,), _F32).at[sel].set(0.0).reshape(8, 1, num_msa // 8)
        gidx = jnp.concatenate([sel, extra_sel]).reshape(nc + nx, 1)

        gb = nc + nx
        msa_g, del_g = pl.pallas_call(
            _gather_kernel,
            grid=(gb // nc,),
            in_specs=[pl.BlockSpec((nc, 1), lambda i: (i, 0)),
                      pl.BlockSpec((num_msa, num_res), lambda i: (0, 0)),
                      pl.BlockSpec((num_msa, num_res), lambda i: (0, 0))],
            out_specs=(pl.BlockSpec((nc, num_res), lambda i: (i, 0)),
                       pl.BlockSpec((nc, num_res), lambda i: (i, 0))),
            out_shape=(jax.ShapeDtypeStruct((gb, num_res), _I32),
                       jax.ShapeDtypeStruct((gb, num_res), _F32)),
        )(gidx, msa, deletion_matrix)
        msa_clust, extra_msa = msa_g[:nc], msa_g[nc:]
        del_clust, extra_del = del_g[:nc], del_g[nc:]

        bert_msa, bert_mask, has_del, del_value = pl.pallas_call(
            _bert_kernel,
            out_shape=(jax.ShapeDtypeStruct((nc, num_res), _I32),
                       jax.ShapeDtypeStruct((nc, num_res), _F32),
                       jax.ShapeDtypeStruct((nc, num_res), _F32),
                       jax.ShapeDtypeStruct((nc, num_res), _F32)),
        )(msa_clust, del_clust, profile_logits_t, gumbel_t,
          mask_pos.astype(_I32), rand_cat, uniform_repl)

        s2d, cnts, dmv = pl.pallas_call(
            _main_kernel,
            grid=(num_msa // nc,),
            in_specs=[pl.BlockSpec((nc, num_res), lambda j: (j, 0)),
                      pl.BlockSpec((nc, num_res), lambda j: (j, 0)),
                      pl.BlockSpec((nc, num_res), lambda j: (0, 0)),
                      pl.BlockSpec((1, 1, nc), lambda j: (j, 0, 0)),
                      pl.BlockSpec((nc, num_res), lambda j: (0, 0))],
            out_specs=(pl.BlockSpec((nc, 23 * num_res), lambda j: (0, 0)),
                       pl.BlockSpec((nc, 1), lambda j: (0, 0)),
                       pl.BlockSpec((nc, num_res), lambda j: (0, 0))),
            out_shape=(jax.ShapeDtypeStruct((nc, 23 * num_res), _F32),
                       jax.ShapeDtypeStruct((nc, 1), _F32),
                       jax.ShapeDtypeStruct((nc, num_res), _F32)),
            scratch_shapes=[pltpu.VMEM((nc, 21 * num_res), jnp.bfloat16),
                            pltpu.VMEM((nc, 23 * num_res), _F32),
                            pltpu.VMEM((nc, num_res), _F32),
                            pltpu.VMEM((nc, 1), _F32)],
        )(msa, deletion_matrix, bert_msa, is_extra, del_clust)

        prof2d, samp2d = pl.pallas_call(
            _feat_kernel,
            grid=(23,),
            in_specs=[pl.BlockSpec((nc, num_res), lambda c: (0, c)),
                      pl.BlockSpec((nc, num_res), lambda c: (0, 0)),
                      pl.BlockSpec((nc, 1), lambda c: (0, 0))],
            out_specs=(pl.BlockSpec((nc, num_res), lambda c: (0, c)),
                       pl.BlockSpec((nc, num_res), lambda c: (0, c))),
            out_shape=(jax.ShapeDtypeStruct((nc, 23 * num_res), _F32),
                       jax.ShapeDtypeStruct((nc, 23 * num_res), _F32)),
        )(s2d, bert_msa, cnts)

        sample_oh = jnp.transpose(samp2d.reshape(nc, 23, num_res), (0, 2, 1))
        cluster_profile = jnp.transpose(prof2d.reshape(nc, 23, num_res), (0, 2, 1))
        msa_feat = jnp.concatenate(
            [sample_oh, has_del[..., None], del_value[..., None],
             cluster_profile, dmv[..., None]], axis=-1)
        msa_feats.append(msa_feat)
        bert_masks.append(bert_mask)
        true_msas.append(msa_clust)
        extra_msas.append(extra_msa)
        extra_dels.append(extra_del)

    target_feat = target_feat_oh
    n_ens = NUM_RECYCLE_C + 1
    return (jnp.stack(msa_feats, 0),
            jnp.broadcast_to(target_feat[None], (n_ens,) + target_feat.shape),
            jnp.stack(bert_masks, 0),
            jnp.stack(true_msas, 0),
            jnp.stack(extra_msas, 0),
            jnp.stack(extra_dels, 0))
